# initial kernel scaffold (unmeasured)
import functools

import jax
import jax.numpy as jnp
from jax import lax
from jax.experimental import pallas as pl
from jax.experimental.pallas import tpu as pltpu

N_DEV = 4
B = 8
H = 8
D = 128
BS = 16
NB = 512
PP = 512
KK = PP * BS
NEG = -1e30


def _qk_body(q_ref, k_ref, bt_ref, lens_ref, p_ref, ml_ref, s_scr):
    my = lax.axis_index("i")
    off = my * PP

    bt = bt_ref[...]
    lens = lens_ref[...]
    jio = lax.broadcasted_iota(jnp.int32, (B, NB, PP), 1)
    pio = lax.broadcasted_iota(jnp.int32, (B, NB, PP), 2)
    hit = jnp.where(
        (bt[:, :, None] == pio + off) & (jio < lens[:, :, None]), 1.0, 0.0
    )
    c = jnp.sum(hit, axis=1)

    prow = lax.broadcasted_iota(jnp.int32, (PP, KK), 0)
    kcol = lax.broadcasted_iota(jnp.int32, (PP, KK), 1)
    expand = jnp.where(prow == kcol // BS, 1.0, 0.0)
    ckeys = lax.dot_general(
        c, expand, (((1,), (0,)), ((), ())),
        preferred_element_type=jnp.float32,
    )

    scale = D ** -0.5
    for h in range(H):
        q_h = q_ref[:, 0, h, :]
        k_h = k_ref[:, :, h, :].reshape(KK, D)
        s_scr[h] = lax.dot_general(
            q_h, k_h, (((1,), (1,)), ((), ())),
            preferred_element_type=jnp.float32,
        ) * scale

    s = s_scr[...]
    sm = jnp.where(ckeys[None] > 0.0, s, NEG)
    m = jnp.max(sm, axis=2, keepdims=True)
    p = jnp.exp(sm - m) * ckeys[None]
    p_ref[...] = p
    ml_ref[0] = jnp.max(sm, axis=2)
    ml_ref[1] = jnp.sum(p, axis=2)


def _pv_body(p_ref, ml_ref, v_ref, out_ref, o_comm, ml_comm,
             send_sems, recv_sems):
    my = lax.axis_index("i")

    for h in range(H):
        v_h = v_ref[:, :, h, :].reshape(KK, D)
        o_comm[my, h] = lax.dot_general(
            p_ref[h], v_h, (((1,), (0,)), ((), ())),
            preferred_element_type=jnp.float32,
        )
    ml_comm[my] = ml_ref[...]

    bar = pltpu.get_barrier_semaphore()
    for dlt in range(1, N_DEV):
        tgt = lax.rem(my + dlt, N_DEV)
        pl.semaphore_signal(bar, inc=1, device_id=(tgt,),
                            device_id_type=pl.DeviceIdType.MESH)
    pl.semaphore_wait(bar, N_DEV - 1)

    sends = []
    for dlt in range(1, N_DEV):
        tgt = lax.rem(my + dlt, N_DEV)
        for t, buf in ((0, o_comm), (1, ml_comm)):
            r = pltpu.make_async_remote_copy(
                src_ref=buf.at[my], dst_ref=buf.at[my],
                send_sem=send_sems.at[dlt - 1, t],
                recv_sem=recv_sems.at[my, t],
                device_id=(tgt,), device_id_type=pl.DeviceIdType.MESH,
            )
            r.start()
            sends.append(r)

    for dlt in range(1, N_DEV):
        src = lax.rem(my + dlt, N_DEV)
        for t, buf in ((0, o_comm), (1, ml_comm)):
            rw = pltpu.make_async_remote_copy(
                src_ref=buf.at[src], dst_ref=buf.at[src],
                send_sem=send_sems.at[dlt - 1, t],
                recv_sem=recv_sems.at[src, t],
                device_id=(src,), device_id_type=pl.DeviceIdType.MESH,
            )
            rw.wait_recv()
    for r in sends:
        r.wait_send()

    mall = ml_comm[:, 0]
    lall = ml_comm[:, 1]
    mg = jnp.max(mall, axis=0, keepdims=True)
    alpha = jnp.exp(mall - mg)
    lg = jnp.sum(alpha * lall, axis=0)
    onum = jnp.sum(alpha[..., None] * o_comm[...], axis=0)
    og = onum / lg[..., None]
    out_ref[:, 0] = jnp.transpose(og, (1, 0, 2))

    @functools.partial(pl.run_scoped, exit_sem=pltpu.SemaphoreType.REGULAR)
    def _(exit_sem):
        for dlt in range(1, N_DEV):
            tgt = lax.rem(my + dlt, N_DEV)
            pl.semaphore_signal(exit_sem, inc=1, device_id=(tgt,),
                                device_id_type=pl.DeviceIdType.MESH)
        pl.semaphore_wait(exit_sem, N_DEV - 1)


def kernel(Q, K, V, bt, lens):
    lens2 = lens.reshape(B, 1)

    p_part, ml = pl.pallas_call(
        _qk_body,
        out_shape=[
            jax.ShapeDtypeStruct((H, B, KK), jnp.float32),
            jax.ShapeDtypeStruct((2, H, B), jnp.float32),
        ],
        in_specs=[
            pl.BlockSpec(memory_space=pltpu.VMEM),
            pl.BlockSpec(memory_space=pltpu.VMEM),
            pl.BlockSpec(memory_space=pltpu.VMEM),
            pl.BlockSpec(memory_space=pltpu.VMEM),
        ],
        out_specs=[
            pl.BlockSpec(memory_space=pltpu.VMEM),
            pl.BlockSpec(memory_space=pltpu.VMEM),
        ],
        scratch_shapes=[pltpu.VMEM((H, B, KK), jnp.float32)],
    )(Q, K, bt, lens2)

    return pl.pallas_call(
        _pv_body,
        out_shape=jax.ShapeDtypeStruct((B, 1, H, D), jnp.float32),
        in_specs=[
            pl.BlockSpec(memory_space=pltpu.VMEM),
            pl.BlockSpec(memory_space=pltpu.VMEM),
            pl.BlockSpec(memory_space=pltpu.VMEM),
        ],
        out_specs=pl.BlockSpec(memory_space=pltpu.VMEM),
        scratch_shapes=[
            pltpu.VMEM((N_DEV, H, B, D), jnp.float32),
            pltpu.VMEM((N_DEV, 2, H, B), jnp.float32),
            pltpu.SemaphoreType.DMA((N_DEV - 1, 2)),
            pltpu.SemaphoreType.DMA((N_DEV, 2)),
        ],
        compiler_params=pltpu.CompilerParams(collective_id=0),
    )(p_part, ml, V)


# baseline (device time: 72189 ns/iter reference)
import functools

import jax
import jax.numpy as jnp
from jax import lax
from jax.experimental import pallas as pl
from jax.experimental.pallas import tpu as pltpu

N_DEV = 4
B = 8
H = 8
D = 128
BS = 16
NB = 512
PP = 512
KK = PP * BS
NEG = -1e30


def _qk_body(q_ref, k_ref, bt_ref, lens_ref, p_ref, ml_ref, s_scr):
    my = lax.axis_index("i")
    off = my * PP

    bt = bt_ref[...]
    lens = lens_ref[...]
    JC = 128
    c = jnp.zeros((B, PP), jnp.float32)
    for j0 in range(0, NB, JC):
        btc = bt[:, j0:j0 + JC]
        jio = lax.broadcasted_iota(jnp.int32, (B, JC, PP), 1) + j0
        pio = lax.broadcasted_iota(jnp.int32, (B, JC, PP), 2)
        hitc = jnp.where(
            (btc[:, :, None] == pio + off) & (jio < lens[:, :, None]),
            1.0, 0.0,
        )
        c = c + jnp.sum(hitc, axis=1)

    prow = lax.broadcasted_iota(jnp.int32, (PP, KK), 0)
    kcol = lax.broadcasted_iota(jnp.int32, (PP, KK), 1)
    expand = jnp.where(prow == kcol // BS, 1.0, 0.0).astype(jnp.bfloat16)
    ckeys = lax.dot_general(
        c.astype(jnp.bfloat16), expand, (((1,), (0,)), ((), ())),
        preferred_element_type=jnp.float32,
    )

    scale = D ** -0.5
    for h in range(H):
        q_h = q_ref[:, 0, h, :]
        k_h = k_ref[:, :, h, :].reshape(KK, D)
        s_scr[h] = lax.dot_general(
            q_h, k_h, (((1,), (1,)), ((), ())),
            preferred_element_type=jnp.float32,
        ) * scale

    s = s_scr[...]
    sm = jnp.where(ckeys[None] > 0.0, s, NEG)
    m = jnp.max(sm, axis=2, keepdims=True)
    p = jnp.exp(sm - m) * ckeys[None]
    p_ref[...] = p
    ml_ref[0] = jnp.max(sm, axis=2)
    ml_ref[1] = jnp.sum(p, axis=2)


def _pv_body(p_ref, ml_ref, v_ref, out_ref, o_comm, ml_comm,
             send_sems, recv_sems):
    my = lax.axis_index("i")

    for h in range(H):
        v_h = v_ref[:, :, h, :].reshape(KK, D)
        o_comm[my, h] = lax.dot_general(
            p_ref[h], v_h, (((1,), (0,)), ((), ())),
            preferred_element_type=jnp.float32,
        )
    ml_comm[my] = ml_ref[...]

    bar = pltpu.get_barrier_semaphore()
    for dlt in range(1, N_DEV):
        tgt = lax.rem(my + dlt, N_DEV)
        pl.semaphore_signal(bar, inc=1, device_id=(tgt,),
                            device_id_type=pl.DeviceIdType.MESH)
    pl.semaphore_wait(bar, N_DEV - 1)

    sends = []
    for dlt in range(1, N_DEV):
        tgt = lax.rem(my + dlt, N_DEV)
        for t, buf in ((0, o_comm), (1, ml_comm)):
            r = pltpu.make_async_remote_copy(
                src_ref=buf.at[my], dst_ref=buf.at[my],
                send_sem=send_sems.at[dlt - 1, t],
                recv_sem=recv_sems.at[my, t],
                device_id=(tgt,), device_id_type=pl.DeviceIdType.MESH,
            )
            r.start()
            sends.append(r)

    for dlt in range(1, N_DEV):
        src = lax.rem(my + dlt, N_DEV)
        for t, buf in ((0, o_comm), (1, ml_comm)):
            rw = pltpu.make_async_remote_copy(
                src_ref=buf.at[src], dst_ref=buf.at[src],
                send_sem=send_sems.at[dlt - 1, t],
                recv_sem=recv_sems.at[src, t],
                device_id=(src,), device_id_type=pl.DeviceIdType.MESH,
            )
            rw.wait_recv()
    for r in sends:
        r.wait_send()

    mall = ml_comm[:, 0]
    lall = ml_comm[:, 1]
    mg = jnp.max(mall, axis=0, keepdims=True)
    alpha = jnp.exp(mall - mg)
    lg = jnp.sum(alpha * lall, axis=0)
    onum = jnp.sum(alpha[..., None] * o_comm[...], axis=0)
    og = onum / lg[..., None]
    out_ref[:, 0] = jnp.transpose(og, (1, 0, 2))

    @functools.partial(pl.run_scoped, exit_sem=pltpu.SemaphoreType.REGULAR)
    def _(exit_sem):
        for dlt in range(1, N_DEV):
            tgt = lax.rem(my + dlt, N_DEV)
            pl.semaphore_signal(exit_sem, inc=1, device_id=(tgt,),
                                device_id_type=pl.DeviceIdType.MESH)
        pl.semaphore_wait(exit_sem, N_DEV - 1)


def kernel(Q, K, V, bt, lens):
    lens2 = lens.reshape(B, 1)

    p_part, ml = pl.pallas_call(
        _qk_body,
        out_shape=[
            jax.ShapeDtypeStruct((H, B, KK), jnp.float32),
            jax.ShapeDtypeStruct((2, H, B), jnp.float32),
        ],
        in_specs=[
            pl.BlockSpec(memory_space=pltpu.VMEM),
            pl.BlockSpec(memory_space=pltpu.VMEM),
            pl.BlockSpec(memory_space=pltpu.VMEM),
            pl.BlockSpec(memory_space=pltpu.VMEM),
        ],
        out_specs=[
            pl.BlockSpec(memory_space=pltpu.VMEM),
            pl.BlockSpec(memory_space=pltpu.VMEM),
        ],
        scratch_shapes=[pltpu.VMEM((H, B, KK), jnp.float32)],
        compiler_params=pltpu.CompilerParams(
            vmem_limit_bytes=60 * 1024 * 1024,
        ),
    )(Q, K, bt, lens2)

    return pl.pallas_call(
        _pv_body,
        out_shape=jax.ShapeDtypeStruct((B, 1, H, D), jnp.float32),
        in_specs=[
            pl.BlockSpec(memory_space=pltpu.VMEM),
            pl.BlockSpec(memory_space=pltpu.VMEM),
            pl.BlockSpec(memory_space=pltpu.VMEM),
        ],
        out_specs=pl.BlockSpec(memory_space=pltpu.VMEM),
        scratch_shapes=[
            pltpu.VMEM((N_DEV, H, B, D), jnp.float32),
            pltpu.VMEM((N_DEV, 2, H, B), jnp.float32),
            pltpu.SemaphoreType.DMA((N_DEV - 1, 2)),
            pltpu.SemaphoreType.DMA((N_DEV, 2)),
        ],
        compiler_params=pltpu.CompilerParams(
            collective_id=0,
            vmem_limit_bytes=60 * 1024 * 1024,
        ),
    )(p_part, ml, V)
